# baseline (speedup 1.0000x reference)
import jax
import jax.numpy as jnp
from jax import lax
from jax.experimental import pallas as pl
from jax.experimental.pallas import tpu as pltpu
from jax.experimental.pallas import tpu_sc as plsc

NC = 2
NS = 16
NW = NC * NS

N = 10000
D = 128
E = 320000
EPW = E // NW
CH = 80
NCH = EPW // CH
SROW = 624
SCP = 104
TAIL = N - NS * SROW

_MESH = plsc.VectorSubcoreMesh(core_axis_name="c", subcore_axis_name="s")


def _pipe2(n, p1_start, p1_wait, p2_start, p2_wait):
    assert n % 2 == 1 and n >= 5
    p1_start(0, 0)
    p1_wait(0)
    p2_start(0, 0)
    p1_start(1, 1)
    p1_wait(1)
    p2_start(1, 1)
    p2_wait(0)
    p1_start(2, 0)

    def body(k, _):
        c = 2 * k
        p1_wait(0)
        p2_start(c, 0)
        p2_wait(1)
        p1_start(c + 1, 1)
        p1_wait(1)
        p2_start(c + 1, 1)
        p2_wait(0)
        p1_start(c + 2, 0)
        return 0

    lax.fori_loop(1, (n - 1) // 2, body, 0)
    p1_wait(0)
    p2_start(n - 1, 0)
    p2_wait(1)
    p2_wait(0)


def _pipeK(n, K, L, p1s, p1w, p2s, p2w):
    assert 0 < L < K <= n
    for c in range(L):
        p1s(c, c % K)
    for j in range(K - L):
        p1w(j % K)
        p2s(j, j % K)
        p1s(j + L, (j + L) % K)
    base = K - L
    groups = (n - base) // K
    while groups > 0 and base + groups * K - 1 + L >= n:
        groups -= 1

    def body(g, _):
        j0 = base + g * K
        for o in range(K):
            p1w((base + o) % K)
            p2s(j0 + o, (base + o) % K)
            p2w((base + o + L) % K)
            p1s(j0 + o + L, (base + o + L) % K)
        return 0

    lax.fori_loop(0, groups, body, 0)
    for j in range(base + groups * K, n):
        p1w(j % K)
        p2s(j, j % K)
        if j + L < n:
            p2w((j + L) % K)
            p1s(j + L, (j + L) % K)
    for j in range(n - K, n):
        p2w(j % K)


def _gather_body(x_hbm, si, di, xs, xd, xsp, idx_v, bufA, bufB, g0, g1,
                 w0, w1):
    cid = lax.axis_index("c")
    sid = lax.axis_index("s")
    wid = sid * NC + cid
    for r in range(SROW // SCP):
        rows = pl.ds(sid * SROW + r * SCP, SCP)
        pltpu.sync_copy(x_hbm.at[rows], bufA)
        pltpu.sync_copy(bufA, xsp.at[rows])

    @pl.when(sid == 0)
    def _():
        rows = pl.ds(NS * SROW, TAIL)
        pltpu.sync_copy(x_hbm.at[rows], bufA.at[pl.ds(0, TAIL)])
        pltpu.sync_copy(bufA.at[pl.ds(0, TAIL)], xsp.at[rows])

    plsc.subcore_barrier()
    base = wid * EPW
    bufs = (bufA.at[pl.ds(0, CH)], bufB.at[pl.ds(0, CH)])
    gsems = (g0, g1)
    wsems = (w0, w1)
    for idx_hbm, out_hbm in ((si, xs), (di, xd)):
        pltpu.sync_copy(idx_hbm.at[wid], idx_v)

        def g_start(c, s):
            pltpu.async_copy(xsp.at[idx_v.at[c]], bufs[s], gsems[s])

        def g_wait(s):
            pltpu.make_async_copy(
                xsp.at[idx_v.at[0]], bufs[s], gsems[s]).wait()

        def w_start(c, s, out_hbm=out_hbm):
            pltpu.async_copy(
                bufs[s], out_hbm.at[pl.ds(base + c * CH, CH)], wsems[s])

        def w_wait(s, out_hbm=out_hbm):
            pltpu.make_async_copy(
                bufs[s], out_hbm.at[pl.ds(base, CH)], wsems[s]).wait()

        _pipe2(NCH, g_start, g_wait, w_start, w_wait)


def _sc_gather(x, si, di):
    eshape = jax.ShapeDtypeStruct((E, D), jnp.float32)
    f = pl.kernel(
        _gather_body,
        out_type=[eshape, eshape],
        mesh=_MESH,
        scratch_types=[
            pltpu.VMEM_SHARED((N, D), jnp.float32),
            pltpu.VMEM((NCH, CH), jnp.int32),
            pltpu.VMEM((SCP, D), jnp.float32),
            pltpu.VMEM((SCP, D), jnp.float32),
            pltpu.SemaphoreType.DMA,
            pltpu.SemaphoreType.DMA,
            pltpu.SemaphoreType.DMA,
            pltpu.SemaphoreType.DMA,
        ],
    )
    return f(x, si, di)


def _scatter_body(msg, didx, zeros_hbm, out, acc, idx_v, b0, b1, b2,
                  r0, r1, r2, s0, s1, s2):
    cid = lax.axis_index("c")
    sid = lax.axis_index("s")
    wid = sid * NC + cid
    base = wid * EPW
    for r in range(SROW // SCP):
        pltpu.sync_copy(zeros_hbm, acc.at[pl.ds(sid * SROW + r * SCP, SCP)])

    @pl.when(sid == 0)
    def _():
        pltpu.sync_copy(zeros_hbm.at[pl.ds(0, TAIL)],
                        acc.at[pl.ds(NS * SROW, TAIL)])

    plsc.subcore_barrier()
    pltpu.sync_copy(didx.at[wid], idx_v)
    bufs = (b0, b1, b2)
    rsems = (r0, r1, r2)
    ssems = (s0, s1, s2)

    def r_start(c, s):
        pltpu.async_copy(msg.at[pl.ds(base + c * CH, CH)], bufs[s], rsems[s])

    def r_wait(s):
        pltpu.make_async_copy(
            msg.at[pl.ds(base, CH)], bufs[s], rsems[s]).wait()

    def s_start(c, s):
        pltpu.async_copy(bufs[s], acc.at[idx_v.at[c]], ssems[s], add=True)

    def s_wait(s):
        pltpu.make_async_copy(bufs[s], acc.at[idx_v.at[0]], ssems[s]).wait()

    _pipeK(NCH, 3, 1, r_start, r_wait, s_start, s_wait)

    plsc.subcore_barrier()
    for r in range(SROW // SCP):
        rows = pl.ds(sid * SROW + r * SCP, SCP)
        pltpu.sync_copy(acc.at[rows], out.at[cid, rows])

    @pl.when(sid == 0)
    def _():
        rows = pl.ds(NS * SROW, TAIL)
        pltpu.sync_copy(acc.at[rows], out.at[cid, rows])


def _sc_scatter(msg, didx, zeros):
    f = pl.kernel(
        _scatter_body,
        out_type=jax.ShapeDtypeStruct((NC, N, D), jnp.float32),
        mesh=_MESH,
        scratch_types=[
            pltpu.VMEM_SHARED((N, D), jnp.float32),
            pltpu.VMEM((NCH, CH), jnp.int32),
        ] + [pltpu.VMEM((CH, D), jnp.float32)] * 3
          + [pltpu.SemaphoreType.DMA] * 6,
    )
    return f(msg, didx, zeros)


_BF = jnp.bfloat16


def _msg_body(xs_ref, xd_ref, w1, b1, w2, b2, lw, lb, out_ref):
    xs = xs_ref[...]
    d = jnp.abs(xs - xd_ref[...]).astype(_BF)
    h = jnp.maximum(
        jnp.dot(d, w1[...], preferred_element_type=jnp.float32) + b1[...], 0.0)
    e = (jnp.dot(h.astype(_BF), w2[...], preferred_element_type=jnp.float32)
         + b2[...])
    out_ref[...] = (
        e + jnp.dot(xs.astype(_BF), lw[...],
                    preferred_element_type=jnp.float32) + lb[...])


_BR = 1000


def _tc_msg(xs, xd, p):
    wspec = pl.BlockSpec((D, D), lambda i: (0, 0))
    bspec = pl.BlockSpec((1, D), lambda i: (0, 0))
    espec = pl.BlockSpec((_BR, D), lambda i: (i, 0))
    return pl.pallas_call(
        _msg_body,
        grid=(E // _BR,),
        in_specs=[espec, espec, wspec, bspec, wspec, bspec, wspec, bspec],
        out_specs=pl.BlockSpec((_BR, D), lambda i: (i, 0)),
        out_shape=jax.ShapeDtypeStruct((E, D), jnp.float32),
    )(xs, xd, p["mlp_W1"].astype(_BF), p["mlp_b1"].reshape(1, D),
      p["mlp_W2"].astype(_BF), p["mlp_b2"].reshape(1, D),
      p["lin_W"].astype(_BF), p["lin_b"].reshape(1, D))


def _final_body(x_ref, p0_ref, p1_ref, p2_ref, lw_ref, lb_ref, pwh_ref,
                pwa_ref, pc_ref, out_ref):
    xb = x_ref[...]
    s = None
    for t, p_ref in enumerate((p0_ref, p1_ref, p2_ref)):
        hx = (jnp.dot(xb, lw_ref[t], preferred_element_type=jnp.float32)
              + lb_ref[t])
        agg = p_ref[0] + p_ref[1]
        o = (jnp.dot(hx, pwh_ref[t], preferred_element_type=jnp.float32)
             + jnp.dot(agg, pwa_ref[t], preferred_element_type=jnp.float32)
             + pc_ref[t])
        s = o if s is None else s + o
    out_ref[...] = s * (1.0 / 3.0)


_BN = 1000


def _tc_final(x, parts, lw, lb, pwh, pwa, pc):
    pspec = pl.BlockSpec((NC, _BN, D), lambda i: (0, i, 0))
    return pl.pallas_call(
        _final_body,
        grid=(N // _BN,),
        in_specs=[
            pl.BlockSpec((_BN, D), lambda i: (i, 0)),
            pspec, pspec, pspec,
            pl.BlockSpec((3, D, D), lambda i: (0, 0, 0)),
            pl.BlockSpec((3, 1, D), lambda i: (0, 0, 0)),
            pl.BlockSpec((3, D, D), lambda i: (0, 0, 0)),
            pl.BlockSpec((3, D, D), lambda i: (0, 0, 0)),
            pl.BlockSpec((3, 1, D), lambda i: (0, 0, 0)),
        ],
        out_specs=pl.BlockSpec((_BN, D), lambda i: (i, 0)),
        out_shape=jax.ShapeDtypeStruct((N, D), jnp.float32),
    )(x, parts[0], parts[1], parts[2], lw, lb, pwh, pwa, pc)


_ETYPES = ("onset", "consecutive", "during")


def kernel(x, edge_index_onset, edge_index_consecutive, edge_index_during,
           params):
    eis = (edge_index_onset, edge_index_consecutive, edge_index_during)
    si = [ei[0].reshape(NW, NCH, CH) for ei in eis]
    di = [ei[1].reshape(NW, NCH, CH) for ei in eis]
    zeros = jnp.zeros((SCP, D), jnp.float32)

    parts = []
    for t, et in enumerate(_ETYPES):
        xs, xd = _sc_gather(x, si[t], di[t])
        msg = _tc_msg(xs, xd, params[et])
        parts.append(_sc_scatter(msg, di[t], zeros))

    lw = jnp.stack([params[et]["lin_W"] for et in _ETYPES])
    lb = jnp.stack([params[et]["lin_b"].reshape(1, D) for et in _ETYPES])
    pwh = jnp.stack([params[et]["proj_W"][:D] for et in _ETYPES])
    pwa = jnp.stack([params[et]["proj_W"][D:] for et in _ETYPES])
    pc = jnp.stack([(params[et]["proj_b"] + params[et]["bias"]).reshape(1, D)
                    for et in _ETYPES])
    return _tc_final(x, parts, lw, lb, pwh, pwa, pc)


# speedup vs baseline: 1.0165x; 1.0165x over previous
import jax
import jax.numpy as jnp
from jax import lax
from jax.experimental import pallas as pl
from jax.experimental.pallas import tpu as pltpu
from jax.experimental.pallas import tpu_sc as plsc

NC = 2
NS = 16
NW = NC * NS

N = 10000
D = 128
E = 320000
EPW = E // NW
CH = 80
NCH = EPW // CH
SROW = 624
SCP = 104
TAIL = N - NS * SROW

_MESH = plsc.VectorSubcoreMesh(core_axis_name="c", subcore_axis_name="s")


def _pipe2(n, p1_start, p1_wait, p2_start, p2_wait):
    assert n % 2 == 1 and n >= 5
    p1_start(0, 0)
    p1_wait(0)
    p2_start(0, 0)
    p1_start(1, 1)
    p1_wait(1)
    p2_start(1, 1)
    p2_wait(0)
    p1_start(2, 0)

    def body(k, _):
        c = 2 * k
        p1_wait(0)
        p2_start(c, 0)
        p2_wait(1)
        p1_start(c + 1, 1)
        p1_wait(1)
        p2_start(c + 1, 1)
        p2_wait(0)
        p1_start(c + 2, 0)
        return 0

    lax.fori_loop(1, (n - 1) // 2, body, 0)
    p1_wait(0)
    p2_start(n - 1, 0)
    p2_wait(1)
    p2_wait(0)


def _gather_body(x_hbm, si, di, xs, xd, xsp, idx_v, bufA, bufB, g0, g1,
                 w0, w1):
    cid = lax.axis_index("c")
    sid = lax.axis_index("s")
    wid = sid * NC + cid
    for r in range(SROW // SCP):
        rows = pl.ds(sid * SROW + r * SCP, SCP)
        pltpu.sync_copy(x_hbm.at[rows], bufA)
        pltpu.sync_copy(bufA, xsp.at[rows])

    @pl.when(sid == 0)
    def _():
        rows = pl.ds(NS * SROW, TAIL)
        pltpu.sync_copy(x_hbm.at[rows], bufA.at[pl.ds(0, TAIL)])
        pltpu.sync_copy(bufA.at[pl.ds(0, TAIL)], xsp.at[rows])

    plsc.subcore_barrier()
    base = wid * EPW
    bufs = (bufA.at[pl.ds(0, CH)], bufB.at[pl.ds(0, CH)])
    gsems = (g0, g1)
    wsems = (w0, w1)
    for idx_hbm, out_hbm in ((si, xs), (di, xd)):
        pltpu.sync_copy(idx_hbm.at[wid], idx_v)

        def g_start(c, s):
            pltpu.async_copy(xsp.at[idx_v.at[c]], bufs[s], gsems[s])

        def g_wait(s):
            pltpu.make_async_copy(
                xsp.at[idx_v.at[0]], bufs[s], gsems[s]).wait()

        def w_start(c, s, out_hbm=out_hbm):
            pltpu.async_copy(
                bufs[s], out_hbm.at[pl.ds(base + c * CH, CH)], wsems[s])

        def w_wait(s, out_hbm=out_hbm):
            pltpu.make_async_copy(
                bufs[s], out_hbm.at[pl.ds(base, CH)], wsems[s]).wait()

        _pipe2(NCH, g_start, g_wait, w_start, w_wait)


def _sc_gather(x, si, di):
    eshape = jax.ShapeDtypeStruct((E, D), jnp.float32)
    f = pl.kernel(
        _gather_body,
        out_type=[eshape, eshape],
        mesh=_MESH,
        scratch_types=[
            pltpu.VMEM_SHARED((N, D), jnp.float32),
            pltpu.VMEM((NCH, CH), jnp.int32),
            pltpu.VMEM((SCP, D), jnp.float32),
            pltpu.VMEM((SCP, D), jnp.float32),
            pltpu.SemaphoreType.DMA,
            pltpu.SemaphoreType.DMA,
            pltpu.SemaphoreType.DMA,
            pltpu.SemaphoreType.DMA,
        ],
    )
    return f(x, si, di)


def _scatter_body(msg, didx, zeros_hbm, out, acc, idx_v, bufA, bufB, fbuf,
                  r0, r1, s0, s1):
    cid = lax.axis_index("c")
    sid = lax.axis_index("s")
    wid = sid * NC + cid
    base = wid * EPW
    pltpu.sync_copy(zeros_hbm, fbuf)
    for r in range(SROW // SCP):
        pltpu.sync_copy(fbuf, acc.at[pl.ds(sid * SROW + r * SCP, SCP)])

    @pl.when(sid == 0)
    def _():
        pltpu.sync_copy(fbuf.at[pl.ds(0, TAIL)],
                        acc.at[pl.ds(NS * SROW, TAIL)])

    plsc.subcore_barrier()
    pltpu.sync_copy(didx.at[wid], idx_v)
    bufs = (bufA, bufB)
    rsems = (r0, r1)
    ssems = (s0, s1)

    def r_start(c, s):
        pltpu.async_copy(msg.at[pl.ds(base + c * CH, CH)], bufs[s], rsems[s])

    def r_wait(s):
        pltpu.make_async_copy(
            msg.at[pl.ds(base, CH)], bufs[s], rsems[s]).wait()

    def s_start(c, s):
        pltpu.async_copy(bufs[s], acc.at[idx_v.at[c]], ssems[s], add=True)

    def s_wait(s):
        pltpu.make_async_copy(bufs[s], acc.at[idx_v.at[0]], ssems[s]).wait()

    _pipe2(NCH, r_start, r_wait, s_start, s_wait)

    plsc.subcore_barrier()
    for r in range(SROW // SCP):
        rows = pl.ds(sid * SROW + r * SCP, SCP)
        pltpu.sync_copy(acc.at[rows], fbuf)
        pltpu.sync_copy(fbuf, out.at[cid, rows])

    @pl.when(sid == 0)
    def _():
        rows = pl.ds(NS * SROW, TAIL)
        pltpu.sync_copy(acc.at[rows], fbuf.at[pl.ds(0, TAIL)])
        pltpu.sync_copy(fbuf.at[pl.ds(0, TAIL)], out.at[cid, rows])


def _sc_scatter(msg, didx, zeros):
    f = pl.kernel(
        _scatter_body,
        out_type=jax.ShapeDtypeStruct((NC, N, D), jnp.float32),
        mesh=_MESH,
        scratch_types=[
            pltpu.VMEM_SHARED((N, D), jnp.float32),
            pltpu.VMEM((NCH, CH), jnp.int32),
            pltpu.VMEM((CH, D), jnp.float32),
            pltpu.VMEM((CH, D), jnp.float32),
            pltpu.VMEM((SCP, D), jnp.float32),
            pltpu.SemaphoreType.DMA,
            pltpu.SemaphoreType.DMA,
            pltpu.SemaphoreType.DMA,
            pltpu.SemaphoreType.DMA,
        ],
    )
    return f(msg, didx, zeros)


_BF = jnp.bfloat16


def _msg_body(xs_ref, xd_ref, w1, b1, w2, b2, lw, lb, out_ref):
    xs = xs_ref[...]
    d = jnp.abs(xs - xd_ref[...]).astype(_BF)
    h = jnp.maximum(
        jnp.dot(d, w1[...], preferred_element_type=jnp.float32) + b1[...], 0.0)
    e = (jnp.dot(h.astype(_BF), w2[...], preferred_element_type=jnp.float32)
         + b2[...])
    out_ref[...] = (
        e + jnp.dot(xs.astype(_BF), lw[...],
                    preferred_element_type=jnp.float32) + lb[...])


_BR = 1000


def _tc_msg(xs, xd, p):
    wspec = pl.BlockSpec((D, D), lambda i: (0, 0))
    bspec = pl.BlockSpec((1, D), lambda i: (0, 0))
    espec = pl.BlockSpec((_BR, D), lambda i: (i, 0))
    return pl.pallas_call(
        _msg_body,
        grid=(E // _BR,),
        in_specs=[espec, espec, wspec, bspec, wspec, bspec, wspec, bspec],
        out_specs=pl.BlockSpec((_BR, D), lambda i: (i, 0)),
        out_shape=jax.ShapeDtypeStruct((E, D), jnp.float32),
    )(xs, xd, p["mlp_W1"].astype(_BF), p["mlp_b1"].reshape(1, D),
      p["mlp_W2"].astype(_BF), p["mlp_b2"].reshape(1, D),
      p["lin_W"].astype(_BF), p["lin_b"].reshape(1, D))


def _final_body(x_ref, p0_ref, p1_ref, p2_ref, lw_ref, lb_ref, pwh_ref,
                pwa_ref, pc_ref, out_ref):
    xb = x_ref[...]
    s = None
    for t, p_ref in enumerate((p0_ref, p1_ref, p2_ref)):
        hx = (jnp.dot(xb, lw_ref[t], preferred_element_type=jnp.float32)
              + lb_ref[t])
        agg = p_ref[0] + p_ref[1]
        o = (jnp.dot(hx, pwh_ref[t], preferred_element_type=jnp.float32)
             + jnp.dot(agg, pwa_ref[t], preferred_element_type=jnp.float32)
             + pc_ref[t])
        s = o if s is None else s + o
    out_ref[...] = s * (1.0 / 3.0)


_BN = 1000


def _tc_final(x, parts, lw, lb, pwh, pwa, pc):
    pspec = pl.BlockSpec((NC, _BN, D), lambda i: (0, i, 0))
    return pl.pallas_call(
        _final_body,
        grid=(N // _BN,),
        in_specs=[
            pl.BlockSpec((_BN, D), lambda i: (i, 0)),
            pspec, pspec, pspec,
            pl.BlockSpec((3, D, D), lambda i: (0, 0, 0)),
            pl.BlockSpec((3, 1, D), lambda i: (0, 0, 0)),
            pl.BlockSpec((3, D, D), lambda i: (0, 0, 0)),
            pl.BlockSpec((3, D, D), lambda i: (0, 0, 0)),
            pl.BlockSpec((3, 1, D), lambda i: (0, 0, 0)),
        ],
        out_specs=pl.BlockSpec((_BN, D), lambda i: (i, 0)),
        out_shape=jax.ShapeDtypeStruct((N, D), jnp.float32),
    )(x, parts[0], parts[1], parts[2], lw, lb, pwh, pwa, pc)


_ETYPES = ("onset", "consecutive", "during")


def kernel(x, edge_index_onset, edge_index_consecutive, edge_index_during,
           params):
    eis = (edge_index_onset, edge_index_consecutive, edge_index_during)
    si = [ei[0].reshape(NW, NCH, CH) for ei in eis]
    di = [ei[1].reshape(NW, NCH, CH) for ei in eis]
    zeros = jnp.zeros((SCP, D), jnp.float32)

    parts = []
    for t, et in enumerate(_ETYPES):
        xs, xd = _sc_gather(x, si[t], di[t])
        msg = _tc_msg(xs, xd, params[et])
        parts.append(_sc_scatter(msg, di[t], zeros))

    lw = jnp.stack([params[et]["lin_W"] for et in _ETYPES])
    lb = jnp.stack([params[et]["lin_b"].reshape(1, D) for et in _ETYPES])
    pwh = jnp.stack([params[et]["proj_W"][:D] for et in _ETYPES])
    pwa = jnp.stack([params[et]["proj_W"][D:] for et in _ETYPES])
    pc = jnp.stack([(params[et]["proj_b"] + params[et]["bias"]).reshape(1, D)
                    for et in _ETYPES])
    return _tc_final(x, parts, lw, lb, pwh, pwa, pc)
